# Initial kernel scaffold; baseline (speedup 1.0000x reference)
import jax
import jax.numpy as jnp
from jax.experimental import pallas as pl


def kernel(x, edge_index, W, att_src, att_dst, bias, W1, b1, W2, b2):
    raise NotImplementedError("write your pallas kernel here")


# speedup vs baseline: 13.2374x; 13.2374x over previous
import jax
import jax.numpy as jnp
from jax import lax
from jax.experimental import pallas as pl
from jax.experimental.pallas import tpu as pltpu
from jax.experimental.pallas import tpu_sc as plsc

N = 10000
NP = 10240
F_IN = 128
HID = 128
HEADS = 4
OUT = 64
RW = HID
E = 320000
EN = E + N
NC = 2
NS = 16
NW = NC * NS
CH = 64
CPT = -(-EN // (NW * CH))
EPT = CPT * CH
ENP = EPT * NW
NPT = NP // NS
ZR = 32
BN = 1024


def _tc1_body(x_ref, w_ref, asrc_ref, adst_ref, hh_ref, as_ref, ad_ref):
    hb = jnp.dot(x_ref[...], w_ref[...], preferred_element_type=jnp.float32)
    for h in range(HEADS):
        hbh = hb[:, h * HID:(h + 1) * HID]
        hh_ref[h] = hbh
        as_ref[h] = jnp.sum(hbh * asrc_ref[h][None, :], axis=1)
        ad_ref[h] = jnp.sum(hbh * adst_ref[h][None, :], axis=1)


def _tc2_body(aggp_ref, denp_ref, bias_ref, w1_ref, b1_ref, w2_ref, b2_ref,
              out_ref):
    t1 = jnp.zeros((BN, HID), jnp.float32)
    for h in range(HEADS):
        den = jnp.sum(denp_ref[:, h], axis=(0, 1)) + 1e-16
        agg = (aggp_ref[0, h] + aggp_ref[1, h]) / den[:, None]
        r = jnp.maximum(agg + bias_ref[h * HID:(h + 1) * HID][None, :], 0.0)
        t1 = t1 + jnp.dot(r, w1_ref[h * HID:(h + 1) * HID, :],
                          preferred_element_type=jnp.float32)
    o1 = jnp.maximum(t1 + b1_ref[...][None, :], 0.0)
    out_ref[...] = (jnp.dot(o1, w2_ref[...], preferred_element_type=jnp.float32)
                    + b2_ref[...][None, :])


def _sc_body(hh2, asT, adT, srcE2, dstE, aggp, denp,
             as_v, ad_v, den_v, src_v, dst_v, rows_v, p_v, zbuf, agg_s, sem):
    c = lax.axis_index("c")
    s = lax.axis_index("s")
    wid = c * NS + s
    tile_base = wid * EPT

    if True:
        def _zb(i, _):
            for v in range(RW // 16):
                zbuf[i, pl.ds(v * 16, 16)] = jnp.zeros((16,), jnp.float32)
            return 0
        lax.fori_loop(0, ZR, _zb, 0)

        def _zd(i, _):
            den_v[pl.ds(i * 16, 16)] = jnp.zeros((16,), jnp.float32)
            return 0

        for h in range(HEADS):
            pltpu.sync_copy(asT.at[h], as_v)
            pltpu.sync_copy(adT.at[h], ad_v)
            lax.fori_loop(0, NP // 16, _zd, 0)
            for b in range(NPT // ZR):
                pltpu.sync_copy(zbuf, agg_s.at[pl.ds(s * NPT + b * ZR, ZR)])
            plsc.subcore_barrier()

            def _edge_chunk(k, _):
                base_e = tile_base + k * CH
                pltpu.sync_copy(srcE2.at[h, pl.ds(base_e, CH)], src_v)
                pltpu.sync_copy(dstE.at[pl.ds(base_e, CH)], dst_v)
                gather = pltpu.async_copy(hh2.at[src_v], rows_v, sem)
                for j in range(CH // 16):
                    sv = src_v[pl.ds(j * 16, 16)] - h * NP
                    dv = dst_v[pl.ds(j * 16, 16)]
                    ee = (plsc.load_gather(as_v, [sv])
                          + plsc.load_gather(ad_v, [dv]))
                    ee = jnp.where(ee >= 0.0, ee, ee * 0.2)
                    pj = jnp.exp(ee)
                    gidx = base_e + j * 16 + lax.iota(jnp.int32, 16)
                    pj = jnp.where(gidx < EN, pj, 0.0)
                    plsc.addupdate_scatter(den_v, [dv], pj)
                    p_v[pl.ds(j * 16, 16)] = pj
                gather.wait()

                def _scale(e, _):
                    pe = plsc.load_gather(p_v, [jnp.full((16,), e, jnp.int32)])
                    for v in range(RW // 16):
                        rows_v[e, pl.ds(v * 16, 16)] = (
                            rows_v[e, pl.ds(v * 16, 16)] * pe)
                    return 0
                lax.fori_loop(0, CH, _scale, 0)
                pltpu.sync_copy(rows_v, agg_s.at[dst_v], add=True)
                return 0
            lax.fori_loop(0, CPT, _edge_chunk, 0)
            plsc.subcore_barrier()

            pltpu.sync_copy(agg_s.at[pl.ds(s * NPT, NPT)],
                            aggp.at[c, h, pl.ds(s * NPT, NPT)])
            pltpu.sync_copy(den_v, denp.at[c, h, s])
            plsc.subcore_barrier()


def kernel(x, edge_index, W, att_src, att_dst, bias, W1, b1, W2, b2):
    loop = jnp.arange(N, dtype=jnp.int32)
    pad = jnp.zeros((ENP - EN,), jnp.int32)
    srcE = jnp.concatenate([edge_index[0].astype(jnp.int32), loop, pad])
    dstE = jnp.concatenate([edge_index[1].astype(jnp.int32), loop, pad])
    srcE2 = srcE[None, :] + (jnp.arange(HEADS, dtype=jnp.int32) * NP)[:, None]
    xp = jnp.pad(x, ((0, NP - N), (0, 0)))

    hh, asT, adT = pl.pallas_call(
        _tc1_body,
        grid=(NP // BN,),
        in_specs=[
            pl.BlockSpec((BN, F_IN), lambda i: (i, 0)),
            pl.BlockSpec((F_IN, HEADS * HID), lambda i: (0, 0)),
            pl.BlockSpec((HEADS, HID), lambda i: (0, 0)),
            pl.BlockSpec((HEADS, HID), lambda i: (0, 0)),
        ],
        out_specs=[
            pl.BlockSpec((HEADS, BN, RW), lambda i: (0, i, 0)),
            pl.BlockSpec((HEADS, BN), lambda i: (0, i)),
            pl.BlockSpec((HEADS, BN), lambda i: (0, i)),
        ],
        out_shape=[
            jax.ShapeDtypeStruct((HEADS, NP, RW), jnp.float32),
            jax.ShapeDtypeStruct((HEADS, NP), jnp.float32),
            jax.ShapeDtypeStruct((HEADS, NP), jnp.float32),
        ],
    )(xp, W, att_src, att_dst)
    hh2 = hh.reshape(HEADS * NP, RW)

    mesh = plsc.VectorSubcoreMesh(core_axis_name="c", subcore_axis_name="s")
    aggp, denp = pl.kernel(
        _sc_body,
        out_type=[
            jax.ShapeDtypeStruct((NC, HEADS, NP, RW), jnp.float32),
            jax.ShapeDtypeStruct((NC, HEADS, NS, NP), jnp.float32),
        ],
        mesh=mesh,
        compiler_params=pltpu.CompilerParams(needs_layout_passes=False),
        scratch_types=[
            pltpu.VMEM((NP,), jnp.float32),
            pltpu.VMEM((NP,), jnp.float32),
            pltpu.VMEM((NP,), jnp.float32),
            pltpu.VMEM((CH,), jnp.int32),
            pltpu.VMEM((CH,), jnp.int32),
            pltpu.VMEM((CH, RW), jnp.float32),
            pltpu.VMEM((CH,), jnp.float32),
            pltpu.VMEM((ZR, RW), jnp.float32),
            pltpu.VMEM_SHARED((NP, RW), jnp.float32),
            pltpu.SemaphoreType.DMA,
        ],
    )(hh2, asT, adT, srcE2, dstE)

    outp = pl.pallas_call(
        _tc2_body,
        grid=(NP // BN,),
        in_specs=[
            pl.BlockSpec((NC, HEADS, BN, RW), lambda i: (0, 0, i, 0)),
            pl.BlockSpec((NC, HEADS, NS, BN), lambda i: (0, 0, 0, i)),
            pl.BlockSpec((HEADS * HID,), lambda i: (0,)),
            pl.BlockSpec((HEADS * HID, HID), lambda i: (0, 0)),
            pl.BlockSpec((HID,), lambda i: (0,)),
            pl.BlockSpec((HID, OUT), lambda i: (0, 0)),
            pl.BlockSpec((OUT,), lambda i: (0,)),
        ],
        out_specs=pl.BlockSpec((BN, OUT), lambda i: (i, 0)),
        out_shape=jax.ShapeDtypeStruct((NP, OUT), jnp.float32),
    )(aggp, denp, bias, W1, b1, W2, b2)

    return outp[:N]
